# baseline (speedup 1.0000x reference)
import functools

import jax
import jax.numpy as jnp
from jax import lax
from jax.experimental import pallas as pl
from jax.experimental.pallas import tpu as pltpu
from jax.experimental.pallas import tpu_sc as plsc

B = 4096
S = 200
D = 64
NCLS = 10
V = 1000000
CH = 100
NCH = S // CH
LANES = 16
GK = 8192
VP = 1000064
PROWS = V * LANES // 128


def _sc_info():
    try:
        info = plsc.get_sparse_core_info()
        return info.num_cores, info.num_subcores
    except Exception:
        return 2, 16


def _project(tt, wtp):
    ng = (V + GK - 1) // GK

    def body(wt_ref, tt_ref, o_hbm, yt0, yt1, sem0, sem1):
        g = pl.program_id(0)
        y = lax.dot_general(wt_ref[...], tt_ref[...],
                            (((1,), (0,)), ((), ())),
                            preferred_element_type=jnp.float32)
        yt = lax.dot_general(y, jnp.eye(LANES, dtype=jnp.float32),
                             (((0,), (0,)), ((), ())),
                             preferred_element_type=jnp.float32)
        bufs = (yt0, yt1)
        sems = (sem0, sem1)
        for par in range(2):
            buf, sem = bufs[par], sems[par]

            @pl.when(lax.rem(g, 2) == par)
            def _():
                @pl.when(g >= 2)
                def _():
                    pltpu.make_async_copy(
                        buf, o_hbm.at[pl.ds((g - 2) * GK, GK)], sem).wait()
                buf[...] = yt
                pltpu.make_async_copy(
                    buf, o_hbm.at[pl.ds(g * GK, GK)], sem).start()

                @pl.when(g == ng - 1)
                def _():
                    pltpu.make_async_copy(
                        buf, o_hbm.at[pl.ds(g * GK, GK)], sem).wait()
                    obuf, osem = bufs[1 - par], sems[1 - par]
                    pltpu.make_async_copy(
                        obuf, o_hbm.at[pl.ds((g - 1) * GK, GK)], osem).wait()

    pvpad = ng * GK
    out = pl.pallas_call(
        body,
        grid=(ng,),
        in_specs=[pl.BlockSpec((LANES, D), lambda g: (0, 0)),
                  pl.BlockSpec((D, GK), lambda g: (0, g))],
        out_specs=pl.BlockSpec(memory_space=pl.ANY),
        out_shape=jax.ShapeDtypeStruct((pvpad, LANES), jnp.float32),
        scratch_shapes=[pltpu.VMEM((GK, LANES), jnp.float32),
                        pltpu.VMEM((GK, LANES), jnp.float32),
                        pltpu.SemaphoreType.DMA,
                        pltpu.SemaphoreType.DMA],
    )(wtp, tt)
    return out


def _pooled_sum(idx2d, p4):
    ncores, nsub = _sc_info()
    nw = ncores * nsub
    bpw = B // nw
    mesh = plsc.VectorSubcoreMesh(
        core_axis_name="c", subcore_axis_name="s",
        num_cores=ncores, num_subcores=nsub)

    nbuf = 8
    nchunks = NCH * bpw
    rows_per_g = nbuf // NCH

    @functools.partial(
        pl.kernel,
        out_type=jax.ShapeDtypeStruct((B, LANES), jnp.float32),
        mesh=mesh,
        compiler_params=pltpu.CompilerParams(use_tc_tiling_on_sc=False),
        scratch_types=[
            pltpu.VMEM((NCH * bpw, CH), jnp.int32),
            [pltpu.VMEM((CH, LANES), jnp.float32) for _ in range(nbuf)],
            pltpu.VMEM((bpw, LANES), jnp.float32),
            [pltpu.SemaphoreType.DMA for _ in range(nbuf)],
        ],
    )
    def k(idx_hbm, p4_hbm, out_hbm, idx_v, bufs, acc_v, sems):
        wid = lax.axis_index("s") * ncores + lax.axis_index("c")
        irow = wid * nchunks
        pltpu.sync_copy(idx_hbm.at[pl.ds(irow, nchunks)], idx_v)

        def fire(j, slot):
            pltpu.async_copy(p4_hbm.at[idx_v.at[j]], bufs[slot], sems[slot])

        def drain(j, slot):
            pltpu.make_async_copy(
                p4_hbm.at[idx_v.at[j]], bufs[slot], sems[slot]).wait()

        def reduce_chunk(buf, acc):
            unroll = 4
            def body(t, acc):
                for u in range(unroll):
                    acc = acc + buf[t * unroll + u, pl.ds(0, LANES)]
                return acc
            return lax.fori_loop(0, CH // unroll, body, acc)

        for slot in range(nbuf):
            fire(slot, slot)

        def step(g, _):
            j0 = g * nbuf
            for r in range(rows_per_g):
                acc = jnp.zeros((LANES,), jnp.float32)
                for h in range(NCH):
                    slot = r * NCH + h
                    j = j0 + slot
                    drain(j, slot)
                    acc = reduce_chunk(bufs[slot], acc)

                    @pl.when(j + nbuf < nchunks)
                    def _():
                        fire(j + nbuf, slot)
                acc_v[g * rows_per_g + r, pl.ds(0, LANES)] = acc
            return 0

        lax.fori_loop(0, nchunks // nbuf, step, 0)
        pltpu.sync_copy(acc_v, out_hbm.at[pl.ds(wid * bpw, bpw)])

    return k(idx2d, p4)


def _tail(pp, b16):
    def body(pp_ref, b_ref, o_ref):
        logits = pp_ref[...][:, :NCLS] * (1.0 / S) + b_ref[...][:, :NCLS]
        m = jnp.max(logits, axis=1, keepdims=True)
        e = jnp.exp(logits - m)
        lse = jnp.log(jnp.sum(e, axis=1, keepdims=True)) + m
        o_ref[...] = logits - lse

    return pl.pallas_call(
        body,
        out_shape=jax.ShapeDtypeStruct((B, NCLS), jnp.float32),
    )(pp, b16)


def kernel(input, table, W, b):
    idx2d = input.astype(jnp.int32).reshape(B * NCH, CH)
    tt = table.T
    wtp = jnp.pad(W.T, ((0, LANES - NCLS), (0, 0)))
    b16 = jnp.pad(b, (0, LANES - NCLS)).reshape(1, LANES)
    p4 = _project(tt, wtp)
    pp = _pooled_sum(idx2d, p4)
    return _tail(pp, b16)


# speedup vs baseline: 1.2123x; 1.2123x over previous
import functools

import jax
import jax.numpy as jnp
from jax import lax
from jax.experimental import pallas as pl
from jax.experimental.pallas import tpu as pltpu
from jax.experimental.pallas import tpu_sc as plsc

B = 4096
S = 200
D = 64
NCLS = 10
V = 1000000
CH = 100
NCH = S // CH
LANES = 16
GK = 8192
VP = 1000064
PROWS = V * LANES // 128


def _sc_info():
    try:
        info = plsc.get_sparse_core_info()
        return info.num_cores, info.num_subcores
    except Exception:
        return 2, 16


def _project(tt, wtp):
    ng = (V + GK - 1) // GK
    rem = VP - (ng - 1) * GK

    def body(wt_ref, tt_ref, o_hbm, y_v, sem):
        g = pl.program_id(0)
        y_v[...] = lax.dot_general(wt_ref[...], tt_ref[...],
                                   (((1,), (0,)), ((), ())),
                                   preferred_element_type=jnp.float32)

        @pl.when(g < ng - 1)
        def _():
            cps = [pltpu.make_async_copy(
                y_v.at[j], o_hbm.at[pl.ds(j * VP + g * GK, GK)], sem)
                for j in range(LANES)]
            for cp in cps:
                cp.start()
            for cp in cps:
                cp.wait()

        @pl.when(g == ng - 1)
        def _():
            cps = [pltpu.make_async_copy(
                y_v.at[j, pl.ds(0, rem)],
                o_hbm.at[pl.ds(j * VP + g * GK, rem)], sem)
                for j in range(LANES)]
            for cp in cps:
                cp.start()
            for cp in cps:
                cp.wait()

    return pl.pallas_call(
        body,
        grid=(ng,),
        in_specs=[pl.BlockSpec((LANES, D), lambda g: (0, 0)),
                  pl.BlockSpec((D, GK), lambda g: (0, g))],
        out_specs=pl.BlockSpec(memory_space=pl.ANY),
        out_shape=jax.ShapeDtypeStruct((LANES * VP,), jnp.float32),
        scratch_shapes=[pltpu.VMEM((LANES, GK), jnp.float32),
                        pltpu.SemaphoreType.DMA],
    )(wtp, tt)


CHK = 1600
NCHK = V // CHK
CUNR = 4


def _pack(ptf):
    ncores, nsub = _sc_info()
    nw = ncores * nsub
    tmax = (NCHK + nw - 1) // nw
    mesh = plsc.VectorSubcoreMesh(
        core_axis_name="c", subcore_axis_name="s",
        num_cores=ncores, num_subcores=nsub)

    @functools.partial(
        pl.kernel,
        out_type=jax.ShapeDtypeStruct((V, LANES), jnp.float32),
        mesh=mesh,
        compiler_params=pltpu.CompilerParams(use_tc_tiling_on_sc=False,
                                             needs_layout_passes=False),
        scratch_types=[
            pltpu.VMEM((LANES, CHK), jnp.float32),
            pltpu.VMEM((CHK, LANES + 1), jnp.float32),
            pltpu.SemaphoreType.DMA,
        ],
    )
    def k(ptf_hbm, out_hbm, ptv, outv, sem):
        wid = lax.axis_index("s") * ncores + lax.axis_index("c")
        lanes_iota = lax.iota(jnp.int32, LANES)
        jcols = [jnp.full((LANES,), j, jnp.int32) for j in range(LANES)]

        def chunk(t, _):
            cid = wid + nw * t

            @pl.when(cid < NCHK)
            def _():
                e0 = cid * CHK
                cps = [pltpu.make_async_copy(
                    ptf_hbm.at[pl.ds(j * VP + e0, CHK)], ptv.at[j], sem)
                    for j in range(LANES)]
                for cp in cps:
                    cp.start()
                for cp in cps:
                    cp.wait()

                def col(c, _):
                    for u in range(CUNR):
                        idx0 = (c * CUNR + u) * LANES + lanes_iota
                        for j in range(LANES):
                            row = ptv[j, pl.ds((c * CUNR + u) * LANES,
                                               LANES)]
                            plsc.store_scatter(outv, [idx0, jcols[j]], row)
                    return 0
                lax.fori_loop(0, CHK // (LANES * CUNR), col, 0)
                pltpu.sync_copy(outv.at[:, pl.ds(0, LANES)],
                                out_hbm.at[pl.ds(e0, CHK)])
            return 0

        lax.fori_loop(0, tmax, chunk, 0)

    return k(ptf)


def _pooled_sum(idx2d, p4):
    ncores, nsub = _sc_info()
    nw = ncores * nsub
    bpw = B // nw
    mesh = plsc.VectorSubcoreMesh(
        core_axis_name="c", subcore_axis_name="s",
        num_cores=ncores, num_subcores=nsub)

    nbuf = 8
    nchunks = NCH * bpw
    rows_per_g = nbuf // NCH

    @functools.partial(
        pl.kernel,
        out_type=jax.ShapeDtypeStruct((B, LANES), jnp.float32),
        mesh=mesh,
        compiler_params=pltpu.CompilerParams(use_tc_tiling_on_sc=False),
        scratch_types=[
            pltpu.VMEM((NCH * bpw, CH), jnp.int32),
            [pltpu.VMEM((CH, LANES), jnp.float32) for _ in range(nbuf)],
            pltpu.VMEM((bpw, LANES), jnp.float32),
            [pltpu.SemaphoreType.DMA for _ in range(nbuf)],
        ],
    )
    def k(idx_hbm, p4_hbm, out_hbm, idx_v, bufs, acc_v, sems):
        wid = lax.axis_index("s") * ncores + lax.axis_index("c")
        irow = wid * nchunks
        pltpu.sync_copy(idx_hbm.at[pl.ds(irow, nchunks)], idx_v)

        def fire(j, slot):
            pltpu.async_copy(p4_hbm.at[idx_v.at[j]], bufs[slot], sems[slot])

        def drain(j, slot):
            pltpu.make_async_copy(
                p4_hbm.at[idx_v.at[j]], bufs[slot], sems[slot]).wait()

        def reduce_chunk(buf, acc):
            unroll = 4
            def body(t, acc):
                for u in range(unroll):
                    acc = acc + buf[t * unroll + u, pl.ds(0, LANES)]
                return acc
            return lax.fori_loop(0, CH // unroll, body, acc)

        for slot in range(nbuf):
            fire(slot, slot)

        def step(g, _):
            j0 = g * nbuf
            for r in range(rows_per_g):
                acc = jnp.zeros((LANES,), jnp.float32)
                for h in range(NCH):
                    slot = r * NCH + h
                    j = j0 + slot
                    drain(j, slot)
                    acc = reduce_chunk(bufs[slot], acc)

                    @pl.when(j + nbuf < nchunks)
                    def _():
                        fire(j + nbuf, slot)
                acc_v[g * rows_per_g + r, pl.ds(0, LANES)] = acc
            return 0

        lax.fori_loop(0, nchunks // nbuf, step, 0)
        pltpu.sync_copy(acc_v, out_hbm.at[pl.ds(wid * bpw, bpw)])

    return k(idx2d, p4)


def _tail(pp, b16):
    def body(pp_ref, b_ref, o_ref):
        logits = pp_ref[...][:, :NCLS] * (1.0 / S) + b_ref[...][:, :NCLS]
        m = jnp.max(logits, axis=1, keepdims=True)
        e = jnp.exp(logits - m)
        lse = jnp.log(jnp.sum(e, axis=1, keepdims=True)) + m
        o_ref[...] = logits - lse

    return pl.pallas_call(
        body,
        out_shape=jax.ShapeDtypeStruct((B, NCLS), jnp.float32),
    )(pp, b16)


def kernel(input, table, W, b):
    idx2d = input.astype(jnp.int32).reshape(B * NCH, CH)
    tt = table.T
    wtp = jnp.pad(W.T, ((0, LANES - NCLS), (0, 0)))
    b16 = jnp.pad(b, (0, LANES - NCLS)).reshape(1, LANES)
    ptf = _project(tt, wtp)
    p4 = _pack(ptf)
    pp = _pooled_sum(idx2d, p4)
    return _tail(pp, b16)


# speedup vs baseline: 1.2771x; 1.0535x over previous
import functools

import jax
import jax.numpy as jnp
from jax import lax
from jax.experimental import pallas as pl
from jax.experimental.pallas import tpu as pltpu
from jax.experimental.pallas import tpu_sc as plsc

B = 4096
S = 200
D = 64
NCLS = 10
V = 1000000
CH = 100
NCH = S // CH
LANES = 16
GK = 8192
VP = 1000064
PROWS = V * LANES // 128


def _sc_info():
    try:
        info = plsc.get_sparse_core_info()
        return info.num_cores, info.num_subcores
    except Exception:
        return 2, 16


def _project(tt, wtp):
    ng = (V + GK - 1) // GK
    rem = VP - (ng - 1) * GK

    def body(wt_ref, tt_ref, o_hbm, y_v, sem):
        g = pl.program_id(0)
        y_v[...] = lax.dot_general(wt_ref[...], tt_ref[...],
                                   (((1,), (0,)), ((), ())),
                                   preferred_element_type=jnp.float32)

        @pl.when(g < ng - 1)
        def _():
            cps = [pltpu.make_async_copy(
                y_v.at[j], o_hbm.at[pl.ds(j * VP + g * GK, GK)], sem)
                for j in range(LANES)]
            for cp in cps:
                cp.start()
            for cp in cps:
                cp.wait()

        @pl.when(g == ng - 1)
        def _():
            cps = [pltpu.make_async_copy(
                y_v.at[j, pl.ds(0, rem)],
                o_hbm.at[pl.ds(j * VP + g * GK, rem)], sem)
                for j in range(LANES)]
            for cp in cps:
                cp.start()
            for cp in cps:
                cp.wait()

    return pl.pallas_call(
        body,
        grid=(ng,),
        in_specs=[pl.BlockSpec((LANES, D), lambda g: (0, 0)),
                  pl.BlockSpec((D, GK), lambda g: (0, g))],
        out_specs=pl.BlockSpec(memory_space=pl.ANY),
        out_shape=jax.ShapeDtypeStruct((LANES * VP,), jnp.float32),
        scratch_shapes=[pltpu.VMEM((LANES, GK), jnp.float32),
                        pltpu.SemaphoreType.DMA],
    )(wtp, tt)


CHK = 1600
NCHK = V // CHK
CUNR = 4


def _pack(ptf):
    ncores, nsub = _sc_info()
    nw = ncores * nsub
    tmax = (NCHK + nw - 1) // nw
    mesh = plsc.VectorSubcoreMesh(
        core_axis_name="c", subcore_axis_name="s",
        num_cores=ncores, num_subcores=nsub)

    @functools.partial(
        pl.kernel,
        out_type=jax.ShapeDtypeStruct((V, LANES), jnp.float32),
        mesh=mesh,
        compiler_params=pltpu.CompilerParams(use_tc_tiling_on_sc=False,
                                             needs_layout_passes=False),
        scratch_types=[
            pltpu.VMEM((LANES, CHK), jnp.float32),
            pltpu.VMEM((LANES, CHK), jnp.float32),
            pltpu.VMEM((CHK, LANES + 1), jnp.float32),
            pltpu.SemaphoreType.DMA,
            pltpu.SemaphoreType.DMA,
        ],
    )
    def k(ptf_hbm, out_hbm, ptv0, ptv1, outv, sem0, sem1):
        wid = lax.axis_index("s") * ncores + lax.axis_index("c")
        lanes_iota = lax.iota(jnp.int32, LANES)
        jcols = [jnp.full((LANES,), j, jnp.int32) for j in range(LANES)]
        bufs = (ptv0, ptv1)
        sems = (sem0, sem1)

        def descs(cid, buf, sem):
            e0 = cid * CHK
            return [pltpu.make_async_copy(
                ptf_hbm.at[pl.ds(j * VP + e0, CHK)], buf.at[j], sem)
                for j in range(LANES)]

        @pl.when(wid < NCHK)
        def _():
            for cp in descs(wid, ptv0, sem0):
                cp.start()

        def chunk(t, _):
            for par in range(2):
                buf, sem = bufs[par], sems[par]
                nbuf_, nsem = bufs[1 - par], sems[1 - par]

                @pl.when(lax.rem(t, 2) == par)
                def _():
                    cid = wid + nw * t

                    @pl.when(cid < NCHK)
                    def _():
                        @pl.when(cid + nw < NCHK)
                        def _():
                            for cp in descs(cid + nw, nbuf_, nsem):
                                cp.start()
                        for cp in descs(cid, buf, sem):
                            cp.wait()

                        def col(c, _):
                            for u in range(CUNR):
                                idx0 = ((c * CUNR + u) * LANES
                                        + lanes_iota)
                                for j in range(LANES):
                                    row = buf[j, pl.ds(
                                        (c * CUNR + u) * LANES, LANES)]
                                    plsc.store_scatter(
                                        outv, [idx0, jcols[j]], row)
                            return 0
                        lax.fori_loop(0, CHK // (LANES * CUNR), col, 0)
                        pltpu.sync_copy(
                            outv.at[:, pl.ds(0, LANES)],
                            out_hbm.at[pl.ds(cid * CHK, CHK)])
            return 0

        lax.fori_loop(0, tmax, chunk, 0)

    return k(ptf)


def _pooled_sum(idx2d, p4):
    ncores, nsub = _sc_info()
    nw = ncores * nsub
    bpw = B // nw
    mesh = plsc.VectorSubcoreMesh(
        core_axis_name="c", subcore_axis_name="s",
        num_cores=ncores, num_subcores=nsub)

    nbuf = 8
    nchunks = NCH * bpw
    rows_per_g = nbuf // NCH

    @functools.partial(
        pl.kernel,
        out_type=jax.ShapeDtypeStruct((B, LANES), jnp.float32),
        mesh=mesh,
        compiler_params=pltpu.CompilerParams(use_tc_tiling_on_sc=False),
        scratch_types=[
            pltpu.VMEM((NCH * bpw, CH), jnp.int32),
            [pltpu.VMEM((CH, LANES), jnp.float32) for _ in range(nbuf)],
            pltpu.VMEM((bpw, LANES), jnp.float32),
            [pltpu.SemaphoreType.DMA for _ in range(nbuf)],
        ],
    )
    def k(idx_hbm, p4_hbm, out_hbm, idx_v, bufs, acc_v, sems):
        wid = lax.axis_index("s") * ncores + lax.axis_index("c")
        irow = wid * nchunks
        pltpu.sync_copy(idx_hbm.at[pl.ds(irow, nchunks)], idx_v)

        def fire(j, slot):
            pltpu.async_copy(p4_hbm.at[idx_v.at[j]], bufs[slot], sems[slot])

        def drain(j, slot):
            pltpu.make_async_copy(
                p4_hbm.at[idx_v.at[j]], bufs[slot], sems[slot]).wait()

        def reduce_chunk(buf, acc):
            unroll = 4
            def body(t, acc):
                for u in range(unroll):
                    acc = acc + buf[t * unroll + u, pl.ds(0, LANES)]
                return acc
            return lax.fori_loop(0, CH // unroll, body, acc)

        for slot in range(nbuf):
            fire(slot, slot)

        def step(g, _):
            j0 = g * nbuf
            for r in range(rows_per_g):
                acc = jnp.zeros((LANES,), jnp.float32)
                for h in range(NCH):
                    slot = r * NCH + h
                    j = j0 + slot
                    drain(j, slot)
                    acc = reduce_chunk(bufs[slot], acc)

                    @pl.when(j + nbuf < nchunks)
                    def _():
                        fire(j + nbuf, slot)
                acc_v[g * rows_per_g + r, pl.ds(0, LANES)] = acc
            return 0

        lax.fori_loop(0, nchunks // nbuf, step, 0)
        pltpu.sync_copy(acc_v, out_hbm.at[pl.ds(wid * bpw, bpw)])

    return k(idx2d, p4)


def _tail(pp, b16):
    def body(pp_ref, b_ref, o_ref):
        logits = pp_ref[...][:, :NCLS] * (1.0 / S) + b_ref[...][:, :NCLS]
        m = jnp.max(logits, axis=1, keepdims=True)
        e = jnp.exp(logits - m)
        lse = jnp.log(jnp.sum(e, axis=1, keepdims=True)) + m
        o_ref[...] = logits - lse

    return pl.pallas_call(
        body,
        out_shape=jax.ShapeDtypeStruct((B, NCLS), jnp.float32),
    )(pp, b16)


def kernel(input, table, W, b):
    idx2d = input.astype(jnp.int32).reshape(B * NCH, CH)
    tt = table.T
    wtp = jnp.pad(W.T, ((0, LANES - NCLS), (0, 0)))
    b16 = jnp.pad(b, (0, LANES - NCLS)).reshape(1, LANES)
    ptf = _project(tt, wtp)
    p4 = _pack(ptf)
    pp = _pooled_sum(idx2d, p4)
    return _tail(pp, b16)


# speedup vs baseline: 1.2889x; 1.0093x over previous
import functools

import jax
import jax.numpy as jnp
from jax import lax
from jax.experimental import pallas as pl
from jax.experimental.pallas import tpu as pltpu
from jax.experimental.pallas import tpu_sc as plsc

B = 4096
S = 200
D = 64
NCLS = 10
V = 1000000
CH = 100
NCH = S // CH
LANES = 16
GK = 8192
VP = 1000064
PROWS = V * LANES // 128


def _sc_info():
    try:
        info = plsc.get_sparse_core_info()
        return info.num_cores, info.num_subcores
    except Exception:
        return 2, 16


def _project(tt, wtp):
    ng = (V + GK - 1) // GK
    rem = VP - (ng - 1) * GK

    def body(wt_ref, tt_ref, o_hbm, y_v, sem):
        g = pl.program_id(0)
        y_v[...] = lax.dot_general(wt_ref[...], tt_ref[...],
                                   (((1,), (0,)), ((), ())),
                                   preferred_element_type=jnp.float32)

        @pl.when(g < ng - 1)
        def _():
            cps = [pltpu.make_async_copy(
                y_v.at[j], o_hbm.at[pl.ds(j * VP + g * GK, GK)], sem)
                for j in range(LANES)]
            for cp in cps:
                cp.start()
            for cp in cps:
                cp.wait()

        @pl.when(g == ng - 1)
        def _():
            cps = [pltpu.make_async_copy(
                y_v.at[j, pl.ds(0, rem)],
                o_hbm.at[pl.ds(j * VP + g * GK, rem)], sem)
                for j in range(LANES)]
            for cp in cps:
                cp.start()
            for cp in cps:
                cp.wait()

    return pl.pallas_call(
        body,
        grid=(ng,),
        in_specs=[pl.BlockSpec((LANES, D), lambda g: (0, 0)),
                  pl.BlockSpec((D, GK), lambda g: (0, g))],
        out_specs=pl.BlockSpec(memory_space=pl.ANY),
        out_shape=jax.ShapeDtypeStruct((LANES * VP,), jnp.float32),
        scratch_shapes=[pltpu.VMEM((LANES, GK), jnp.float32),
                        pltpu.SemaphoreType.DMA],
    )(wtp, tt)


CHK = 1600
NCHK = V // CHK
CUNR = 8


def _pack(ptf):
    ncores, nsub = _sc_info()
    nw = ncores * nsub
    tmax = (NCHK + nw - 1) // nw
    mesh = plsc.VectorSubcoreMesh(
        core_axis_name="c", subcore_axis_name="s",
        num_cores=ncores, num_subcores=nsub)

    @functools.partial(
        pl.kernel,
        out_type=jax.ShapeDtypeStruct((V, LANES), jnp.float32),
        mesh=mesh,
        compiler_params=pltpu.CompilerParams(use_tc_tiling_on_sc=False,
                                             needs_layout_passes=False),
        scratch_types=[
            pltpu.VMEM((LANES, CHK), jnp.float32),
            pltpu.VMEM((LANES, CHK), jnp.float32),
            pltpu.VMEM((CHK, LANES + 1), jnp.float32),
            pltpu.SemaphoreType.DMA,
            pltpu.SemaphoreType.DMA,
        ],
    )
    def k(ptf_hbm, out_hbm, ptv0, ptv1, outv, sem0, sem1):
        wid = lax.axis_index("s") * ncores + lax.axis_index("c")
        lanes_iota = lax.iota(jnp.int32, LANES)
        jcols = [jnp.full((LANES,), j, jnp.int32) for j in range(LANES)]
        bufs = (ptv0, ptv1)
        sems = (sem0, sem1)

        def descs(cid, buf, sem):
            e0 = cid * CHK
            return [pltpu.make_async_copy(
                ptf_hbm.at[pl.ds(j * VP + e0, CHK)], buf.at[j], sem)
                for j in range(LANES)]

        @pl.when(wid < NCHK)
        def _():
            for cp in descs(wid, ptv0, sem0):
                cp.start()

        def chunk(t, _):
            for par in range(2):
                buf, sem = bufs[par], sems[par]
                nbuf_, nsem = bufs[1 - par], sems[1 - par]

                @pl.when(lax.rem(t, 2) == par)
                def _():
                    cid = wid + nw * t

                    @pl.when(cid < NCHK)
                    def _():
                        @pl.when(cid + nw < NCHK)
                        def _():
                            for cp in descs(cid + nw, nbuf_, nsem):
                                cp.start()
                        for cp in descs(cid, buf, sem):
                            cp.wait()

                        def col(c, _):
                            for u in range(CUNR):
                                idx0 = ((c * CUNR + u) * LANES
                                        + lanes_iota)
                                for j in range(LANES):
                                    row = buf[j, pl.ds(
                                        (c * CUNR + u) * LANES, LANES)]
                                    plsc.store_scatter(
                                        outv, [idx0, jcols[j]], row)
                            return 0
                        lax.fori_loop(0, CHK // (LANES * CUNR), col, 0)
                        pltpu.sync_copy(
                            outv.at[:, pl.ds(0, LANES)],
                            out_hbm.at[pl.ds(cid * CHK, CHK)])
            return 0

        lax.fori_loop(0, tmax, chunk, 0)

    return k(ptf)


def _pooled_sum(idx2d, p4):
    ncores, nsub = _sc_info()
    nw = ncores * nsub
    bpw = B // nw
    mesh = plsc.VectorSubcoreMesh(
        core_axis_name="c", subcore_axis_name="s",
        num_cores=ncores, num_subcores=nsub)

    nbuf = 8
    nchunks = NCH * bpw
    rows_per_g = nbuf // NCH

    @functools.partial(
        pl.kernel,
        out_type=jax.ShapeDtypeStruct((B, LANES), jnp.float32),
        mesh=mesh,
        compiler_params=pltpu.CompilerParams(use_tc_tiling_on_sc=False),
        scratch_types=[
            pltpu.VMEM((NCH * bpw, CH), jnp.int32),
            [pltpu.VMEM((CH, LANES), jnp.float32) for _ in range(nbuf)],
            pltpu.VMEM((bpw, LANES), jnp.float32),
            [pltpu.SemaphoreType.DMA for _ in range(nbuf)],
        ],
    )
    def k(idx_hbm, p4_hbm, out_hbm, idx_v, bufs, acc_v, sems):
        wid = lax.axis_index("s") * ncores + lax.axis_index("c")
        irow = wid * nchunks
        pltpu.sync_copy(idx_hbm.at[pl.ds(irow, nchunks)], idx_v)

        def fire(j, slot):
            pltpu.async_copy(p4_hbm.at[idx_v.at[j]], bufs[slot], sems[slot])

        def drain(j, slot):
            pltpu.make_async_copy(
                p4_hbm.at[idx_v.at[j]], bufs[slot], sems[slot]).wait()

        def reduce_chunk(buf, acc):
            unroll = 4
            def body(t, acc):
                for u in range(unroll):
                    acc = acc + buf[t * unroll + u, pl.ds(0, LANES)]
                return acc
            return lax.fori_loop(0, CH // unroll, body, acc)

        for slot in range(nbuf):
            fire(slot, slot)

        def step(g, _):
            j0 = g * nbuf
            for r in range(rows_per_g):
                acc = jnp.zeros((LANES,), jnp.float32)
                for h in range(NCH):
                    slot = r * NCH + h
                    j = j0 + slot
                    drain(j, slot)
                    acc = reduce_chunk(bufs[slot], acc)

                    @pl.when(j + nbuf < nchunks)
                    def _():
                        fire(j + nbuf, slot)
                acc_v[g * rows_per_g + r, pl.ds(0, LANES)] = acc
            return 0

        lax.fori_loop(0, nchunks // nbuf, step, 0)
        pltpu.sync_copy(acc_v, out_hbm.at[pl.ds(wid * bpw, bpw)])

    return k(idx2d, p4)


def _tail(pp, b16):
    def body(pp_ref, b_ref, o_ref):
        logits = pp_ref[...][:, :NCLS] * (1.0 / S) + b_ref[...][:, :NCLS]
        m = jnp.max(logits, axis=1, keepdims=True)
        e = jnp.exp(logits - m)
        lse = jnp.log(jnp.sum(e, axis=1, keepdims=True)) + m
        o_ref[...] = logits - lse

    return pl.pallas_call(
        body,
        out_shape=jax.ShapeDtypeStruct((B, NCLS), jnp.float32),
    )(pp, b16)


def kernel(input, table, W, b):
    idx2d = input.astype(jnp.int32).reshape(B * NCH, CH)
    tt = table.T
    wtp = jnp.pad(W.T, ((0, LANES - NCLS), (0, 0)))
    b16 = jnp.pad(b, (0, LANES - NCLS)).reshape(1, LANES)
    ptf = _project(tt, wtp)
    p4 = _pack(ptf)
    pp = _pooled_sum(idx2d, p4)
    return _tail(pp, b16)
